# baseline (speedup 1.0000x reference)
import functools

import jax
import jax.numpy as jnp
from jax import lax
from jax.experimental import pallas as pl
from jax.experimental.pallas import tpu as pltpu
from jax.experimental.pallas import tpu_sc as plsc

_B, _L, _D = 1024, 200, 32
_NC, _NS = 2, 16
_NW = _NC * _NS
_RPW = _B // _NW
_T = 256
_W = _RPW * _L
_SC = 640
_NSC = _W // _SC
_GC = 128
_NGC = _SC // _GC


def _table_kernel(w1_ref, b1_ref, w2_ref, b2_ref, t2_ref, t1_ref):
    i = pl.program_id(0)

    @pl.when(i == 0)
    def _():
        a = lax.broadcasted_iota(jnp.int32, (_T, 1), 0).astype(jnp.float32)
        h = jnp.maximum(a * w1_ref[...] + b1_ref[...], 0.0)
        t1_ref[...] = (
            jnp.dot(h, w2_ref[...], preferred_element_type=jnp.float32)
            + b2_ref[...]
        )

    t1 = t1_ref[...]
    for j in range(16):
        row = t1_ref[pl.ds(i * 16 + j, 1), :]
        t2_ref[j * _T:(j + 1) * _T, :] = t1 + row


def _build_table(W1, b1, W2, b2):
    return pl.pallas_call(
        _table_kernel,
        grid=(16,),
        in_specs=[
            pl.BlockSpec((1, _D), lambda i: (0, 0)),
            pl.BlockSpec((1, _D), lambda i: (0, 0)),
            pl.BlockSpec((_D, _D), lambda i: (0, 0)),
            pl.BlockSpec((1, _D), lambda i: (0, 0)),
        ],
        out_specs=pl.BlockSpec((16 * _T, _D), lambda i: (i, 0)),
        out_shape=jax.ShapeDtypeStruct((_T * _T, _D), jnp.float32),
        scratch_shapes=[pltpu.VMEM((_T, _D), jnp.float32)],
    )(W1, b1.reshape(1, _D), W2, b2.reshape(1, _D))


_OFFS = tuple(c * 16 for c in range(12)) + (184,)


@functools.partial(
    pl.kernel,
    out_type=(
        jax.ShapeDtypeStruct((_B * _L, _D), jnp.float32),
        jax.ShapeDtypeStruct((_B * _L, _D), jnp.float32),
    ),
    mesh=plsc.VectorSubcoreMesh(core_axis_name="c", subcore_axis_name="s"),
    compiler_params=pltpu.CompilerParams(
        needs_layout_passes=False, use_tc_tiling_on_sc=False),
    scratch_types=[
        pltpu.VMEM((_W,), jnp.int32),
        pltpu.VMEM((_W,), jnp.int32),
        pltpu.VMEM((_W,), jnp.int32),
        pltpu.VMEM((_W,), jnp.int32),
        pltpu.VMEM((1024,), jnp.int32),
        pltpu.VMEM((1024,), jnp.int32),
        pltpu.VMEM((_SC, _D), jnp.float32),
        pltpu.VMEM((_SC, _D), jnp.float32),
        pltpu.SemaphoreType.DMA,
        pltpu.SemaphoreType.DMA,
        pltpu.SemaphoreType.DMA,
    ],
)
def _sc_encode(src_hbm, dst_hbm, t2_hbm, out_src_hbm, out_dst_hbm,
               sids_v, dids_v, ps_v, pd_v, hs_v, hd_v, buf0, buf1,
               sem, gsem0, gsem1):
    wid = lax.axis_index("s") * _NC + lax.axis_index("c")
    base = wid * _W
    cp_s = pltpu.async_copy(src_hbm.at[pl.ds(base, _W)], sids_v, sem)
    cp_d = pltpu.async_copy(dst_hbm.at[pl.ds(base, _W)], dids_v, sem)
    cp_s.wait()
    cp_d.wait()

    zero16 = jnp.zeros((16,), jnp.int32)
    one16 = jnp.ones((16,), jnp.int32)
    tail = lax.broadcasted_iota(jnp.int32, (16,), 0) >= 8

    def row_body(r, carry):
        rb = r * _L
        for k in range(64):
            hs_v[pl.ds(k * 16, 16)] = zero16
            hd_v[pl.ds(k * 16, 16)] = zero16
        for i, off in enumerate(_OFFS):
            m = tail if i == 12 else None
            plsc.addupdate_scatter(
                hs_v, [sids_v[pl.ds(rb + off, 16)]], one16, mask=m)
            plsc.addupdate_scatter(
                hd_v, [dids_v[pl.ds(rb + off, 16)]], one16, mask=m)
        for off in _OFFS:
            ids = sids_v[pl.ds(rb + off, 16)]
            p = plsc.load_gather(hs_v, [ids]) * _T + plsc.load_gather(hd_v, [ids])
            ps_v[pl.ds(rb + off, 16)] = jnp.where(ids == 0, zero16, p)
            idd = dids_v[pl.ds(rb + off, 16)]
            q = plsc.load_gather(hs_v, [idd]) * _T + plsc.load_gather(hd_v, [idd])
            pd_v[pl.ds(rb + off, 16)] = jnp.where(idd == 0, zero16, q)
        return carry

    lax.fori_loop(0, _RPW, row_body, 0)

    bufs = (buf0, buf1)
    gsems = (gsem0, gsem1)
    for pv, out_hbm in ((ps_v, out_src_hbm), (pd_v, out_dst_hbm)):
        gath = {}

        def issue(k, pv=pv):
            s = k % 2
            gath[k] = [
                pltpu.async_copy(
                    t2_hbm.at[pv.at[pl.ds(k * _SC + j * _GC, _GC)]],
                    bufs[s].at[pl.ds(j * _GC, _GC)], gsems[s])
                for j in range(_NGC)
            ]

        issue(0)
        issue(1)
        for k in range(_NSC):
            s = k % 2
            for cp in gath.pop(k):
                cp.wait()
            pltpu.sync_copy(bufs[s], out_hbm.at[pl.ds(base + k * _SC, _SC)])
            if k + 2 < _NSC:
                issue(k + 2)


def kernel(src_ids, dst_ids, W1, b1, W2, b2):
    t2 = _build_table(W1, b1, W2, b2)
    out_src, out_dst = _sc_encode(
        src_ids.reshape(_B * _L), dst_ids.reshape(_B * _L), t2)
    return (out_src.reshape(_B, _L, _D), out_dst.reshape(_B, _L, _D))


# speedup vs baseline: 8.4245x; 8.4245x over previous
import functools

import jax
import jax.numpy as jnp
from jax import lax
from jax.experimental import pallas as pl
from jax.experimental.pallas import tpu as pltpu
from jax.experimental.pallas import tpu_sc as plsc

_B, _L, _D = 1024, 200, 32
_NC, _NS = 2, 16
_NW = _NC * _NS
_RPW = _B // _NW
_T = 256
_ST = 208
_T2R = 201 * _ST
_T2S = _T2R // _NS
_W = _RPW * _L
_SC = 400
_NSC = _W // _SC
_GC = 80
_NGC = _SC // _GC


def _table_kernel(w1_ref, b1_ref, w2_ref, b2_ref, t2_ref, t1_ref):
    i = pl.program_id(0)

    @pl.when(i == 0)
    def _():
        a = lax.broadcasted_iota(jnp.int32, (_T, 1), 0).astype(jnp.float32)
        h = jnp.maximum(a * w1_ref[...] + b1_ref[...], 0.0)
        t1_ref[...] = (
            jnp.dot(h, w2_ref[...], preferred_element_type=jnp.float32)
            + b2_ref[...]
        )

    t2_ref[...] = t1_ref[pl.ds(0, _ST), :] + t1_ref[pl.ds(i, 1), :]


def _build_table(W1, b1, W2, b2):
    return pl.pallas_call(
        _table_kernel,
        grid=(201,),
        in_specs=[
            pl.BlockSpec((1, _D), lambda i: (0, 0)),
            pl.BlockSpec((1, _D), lambda i: (0, 0)),
            pl.BlockSpec((_D, _D), lambda i: (0, 0)),
            pl.BlockSpec((1, _D), lambda i: (0, 0)),
        ],
        out_specs=pl.BlockSpec((_ST, _D), lambda i: (i, 0)),
        out_shape=jax.ShapeDtypeStruct((_T2R, _D), jnp.float32),
        scratch_shapes=[pltpu.VMEM((_T, _D), jnp.float32)],
    )(W1, b1.reshape(1, _D), W2, b2.reshape(1, _D))


_OFFS = tuple(c * 16 for c in range(12)) + (184,)


@functools.partial(
    pl.kernel,
    out_type=(
        jax.ShapeDtypeStruct((_B * _L, _D), jnp.float32),
        jax.ShapeDtypeStruct((_B * _L, _D), jnp.float32),
    ),
    mesh=plsc.VectorSubcoreMesh(core_axis_name="c", subcore_axis_name="s"),
    compiler_params=pltpu.CompilerParams(
        needs_layout_passes=False, use_tc_tiling_on_sc=False),
    scratch_types=[
        pltpu.VMEM((_W,), jnp.int32),
        pltpu.VMEM((_W,), jnp.int32),
        pltpu.VMEM((1024,), jnp.int32),
        pltpu.VMEM((1024,), jnp.int32),
        pltpu.VMEM((_SC, _D), jnp.float32),
        pltpu.VMEM((_SC, _D), jnp.float32),
        pltpu.VMEM_SHARED((_T2R, _D), jnp.float32),
        pltpu.SemaphoreType.DMA,
        pltpu.SemaphoreType.DMA,
        pltpu.SemaphoreType.DMA,
        pltpu.SemaphoreType.DMA,
    ],
)
def _sc_encode(src_hbm, dst_hbm, t2_hbm, out_src_hbm, out_dst_hbm,
               sids_v, dids_v, hs_v, hd_v, buf0, buf1, t2_sh,
               sem, gsem0, gsem1, tsem):
    sid = lax.axis_index("s")
    wid = sid * _NC + lax.axis_index("c")
    base = wid * _W
    cp_t = pltpu.async_copy(
        t2_hbm.at[pl.ds(sid * _T2S, _T2S)],
        t2_sh.at[pl.ds(sid * _T2S, _T2S)], tsem)
    cp_s = pltpu.async_copy(src_hbm.at[pl.ds(base, _W)], sids_v, sem)
    cp_d = pltpu.async_copy(dst_hbm.at[pl.ds(base, _W)], dids_v, sem)
    cp_s.wait()
    cp_d.wait()
    cp_t.wait()
    plsc.subcore_barrier()

    zero16 = jnp.zeros((16,), jnp.int32)
    one16 = jnp.ones((16,), jnp.int32)
    iota16 = lax.broadcasted_iota(jnp.int32, (16,), 0)
    tail = iota16 >= 8

    def row_body(r, carry):
        rb = r * _L
        for k in range(64):
            hs_v[pl.ds(k * 16, 16)] = zero16
            hd_v[pl.ds(k * 16, 16)] = zero16
        for i, off in enumerate(_OFFS):
            m = tail if i == 12 else None
            plsc.addupdate_scatter(
                hs_v, [sids_v[pl.ds(rb + off, 16)]], one16, mask=m)
            plsc.addupdate_scatter(
                hd_v, [dids_v[pl.ds(rb + off, 16)]], one16, mask=m)
        for i, off in enumerate(_OFFS):
            ids = sids_v[pl.ds(rb + off, 16)]
            p = plsc.load_gather(hs_v, [ids]) * _ST + plsc.load_gather(hd_v, [ids])
            p = jnp.where(ids == 0, zero16, p)
            idd = dids_v[pl.ds(rb + off, 16)]
            q = plsc.load_gather(hs_v, [idd]) * _ST + plsc.load_gather(hd_v, [idd])
            q = jnp.where(idd == 0, zero16, q)
            if i < 12:
                sids_v[pl.ds(rb + off, 16)] = p
                dids_v[pl.ds(rb + off, 16)] = q
            else:
                pos = iota16 + (rb + off)
                plsc.store_scatter(sids_v, [pos], p, mask=tail)
                plsc.store_scatter(dids_v, [pos], q, mask=tail)
        return carry

    lax.fori_loop(0, _RPW, row_body, 0)

    bufs = (buf0, buf1)
    gsems = (gsem0, gsem1)
    for pv, out_hbm in ((sids_v, out_src_hbm), (dids_v, out_dst_hbm)):
        gath = {}

        def issue(k, pv=pv):
            s = k % 2
            gath[k] = [
                pltpu.async_copy(
                    t2_sh.at[pv.at[pl.ds(k * _SC + j * _GC, _GC)]],
                    bufs[s].at[pl.ds(j * _GC, _GC)], gsems[s])
                for j in range(_NGC)
            ]

        issue(0)
        issue(1)
        for k in range(_NSC):
            s = k % 2
            for cp in gath.pop(k):
                cp.wait()
            pltpu.sync_copy(bufs[s], out_hbm.at[pl.ds(base + k * _SC, _SC)])
            if k + 2 < _NSC:
                issue(k + 2)


def kernel(src_ids, dst_ids, W1, b1, W2, b2):
    t2 = _build_table(W1, b1, W2, b2)
    out_src, out_dst = _sc_encode(
        src_ids.reshape(_B * _L), dst_ids.reshape(_B * _L), t2)
    return (out_src.reshape(_B, _L, _D), out_dst.reshape(_B, _L, _D))


# speedup vs baseline: 25.4806x; 3.0246x over previous
import functools

import jax
import jax.numpy as jnp
from jax import lax
from jax.experimental import pallas as pl
from jax.experimental.pallas import tpu as pltpu
from jax.experimental.pallas import tpu_sc as plsc

_B, _L, _D = 1024, 200, 32
_NC, _NS = 2, 16
_NW = _NC * _NS
_RPW = _B // _NW
_T = 256
_W = _RPW * _L
_LB = 8

_OFFS = tuple(c * 16 for c in range(12)) + (184,)


@functools.partial(
    pl.kernel,
    out_type=(
        jax.ShapeDtypeStruct((_B * _L,), jnp.int32),
        jax.ShapeDtypeStruct((_B * _L,), jnp.int32),
    ),
    mesh=plsc.VectorSubcoreMesh(core_axis_name="c", subcore_axis_name="s"),
    compiler_params=pltpu.CompilerParams(
        needs_layout_passes=False, use_tc_tiling_on_sc=False),
    scratch_types=[
        pltpu.VMEM((_W,), jnp.int32),
        pltpu.VMEM((_W,), jnp.int32),
        pltpu.VMEM((1024,), jnp.int32),
        pltpu.VMEM((1024,), jnp.int32),
        pltpu.SemaphoreType.DMA,
    ],
)
def _sc_counts(src_hbm, dst_hbm, ps_hbm, pd_hbm,
               sids_v, dids_v, hs_v, hd_v, sem):
    wid = lax.axis_index("s") * _NC + lax.axis_index("c")
    base = wid * _W
    cp_s = pltpu.async_copy(src_hbm.at[pl.ds(base, _W)], sids_v, sem)
    cp_d = pltpu.async_copy(dst_hbm.at[pl.ds(base, _W)], dids_v, sem)
    cp_s.wait()
    cp_d.wait()

    zero16 = jnp.zeros((16,), jnp.int32)
    one16 = jnp.ones((16,), jnp.int32)
    iota16 = lax.broadcasted_iota(jnp.int32, (16,), 0)
    tail = iota16 >= 8

    def row_body(r, carry):
        rb = r * _L
        for k in range(64):
            hs_v[pl.ds(k * 16, 16)] = zero16
            hd_v[pl.ds(k * 16, 16)] = zero16
        for i, off in enumerate(_OFFS):
            m = tail if i == 12 else None
            plsc.addupdate_scatter(
                hs_v, [sids_v[pl.ds(rb + off, 16)]], one16, mask=m)
            plsc.addupdate_scatter(
                hd_v, [dids_v[pl.ds(rb + off, 16)]], one16, mask=m)
        for i, off in enumerate(_OFFS):
            ids = sids_v[pl.ds(rb + off, 16)]
            p = plsc.load_gather(hs_v, [ids]) * _T + plsc.load_gather(hd_v, [ids])
            p = jnp.where(ids == 0, zero16, p)
            idd = dids_v[pl.ds(rb + off, 16)]
            q = plsc.load_gather(hs_v, [idd]) * _T + plsc.load_gather(hd_v, [idd])
            q = jnp.where(idd == 0, zero16, q)
            if i < 12:
                sids_v[pl.ds(rb + off, 16)] = p
                dids_v[pl.ds(rb + off, 16)] = q
            else:
                pos = iota16 + (rb + off)
                plsc.store_scatter(sids_v, [pos], p, mask=tail)
                plsc.store_scatter(dids_v, [pos], q, mask=tail)
        return carry

    lax.fori_loop(0, _RPW, row_body, 0)

    pltpu.sync_copy(sids_v, ps_hbm.at[pl.ds(base, _W)])
    pltpu.sync_copy(dids_v, pd_hbm.at[pl.ds(base, _W)])


def _expand_kernel(ps_ref, pd_ref, w1t_ref, b1t_ref, w2_ref, b2t_ref,
                   os_ref, od_ref, t1t_ref):
    i = pl.program_id(0)

    @pl.when(i == 0)
    def _():
        a = lax.broadcasted_iota(jnp.int32, (1, _T), 1).astype(jnp.float32)
        x = w1t_ref[...] * a + b1t_ref[...]
        h = jnp.maximum(x, 0.0)
        t1t_ref[...] = lax.dot_general(
            w2_ref[...], h, (((0,), (0,)), ((), ())),
            preferred_element_type=jnp.float32,
        ) + b2t_ref[...]

    t1t = t1t_ref[...]
    iota_v = lax.broadcasted_iota(jnp.int32, (_T, _B), 0)
    one = jnp.float32(1.0)
    zero = jnp.float32(0.0)
    for pair_ref, out_ref in ((ps_ref, os_ref), (pd_ref, od_ref)):
        for j in range(_LB):
            pjr = pair_ref[pl.ds(j, 1), :]
            c1 = pjr // _T
            c2 = pjr - c1 * _T
            oh = (jnp.where(iota_v == c1, one, zero)
                  + jnp.where(iota_v == c2, one, zero))
            blk = lax.dot_general(
                t1t, oh, (((1,), (0,)), ((), ())),
                preferred_element_type=jnp.float32,
            )
            out_ref[j, :, :] = blk


def _tc_expand(ps_t, pd_t, W1, b1, W2, b2):
    return pl.pallas_call(
        _expand_kernel,
        grid=(_L // _LB,),
        in_specs=[
            pl.BlockSpec((_LB, _B), lambda i: (i, 0)),
            pl.BlockSpec((_LB, _B), lambda i: (i, 0)),
            pl.BlockSpec((_D, 1), lambda i: (0, 0)),
            pl.BlockSpec((_D, 1), lambda i: (0, 0)),
            pl.BlockSpec((_D, _D), lambda i: (0, 0)),
            pl.BlockSpec((_D, 1), lambda i: (0, 0)),
        ],
        out_specs=[
            pl.BlockSpec((_LB, _D, _B), lambda i: (i, 0, 0)),
            pl.BlockSpec((_LB, _D, _B), lambda i: (i, 0, 0)),
        ],
        out_shape=[
            jax.ShapeDtypeStruct((_L, _D, _B), jnp.float32),
            jax.ShapeDtypeStruct((_L, _D, _B), jnp.float32),
        ],
        scratch_shapes=[pltpu.VMEM((_D, _T), jnp.float32)],
    )(ps_t, pd_t, W1.reshape(_D, 1), b1.reshape(_D, 1), W2,
      b2.reshape(_D, 1))


def kernel(src_ids, dst_ids, W1, b1, W2, b2):
    ps, pd = _sc_counts(src_ids.reshape(_B * _L), dst_ids.reshape(_B * _L))
    ps_t = ps.reshape(_B, _L).T
    pd_t = pd.reshape(_B, _L).T
    os3, od3 = _tc_expand(ps_t, pd_t, W1, b1, W2, b2)
    return (jnp.transpose(os3, (2, 0, 1)), jnp.transpose(od3, (2, 0, 1)))


# speedup vs baseline: 34.8206x; 1.3666x over previous
import functools

import jax
import jax.numpy as jnp
from jax import lax
from jax.experimental import pallas as pl
from jax.experimental.pallas import tpu as pltpu
from jax.experimental.pallas import tpu_sc as plsc

_B, _L, _D = 1024, 200, 32
_NC, _NS = 2, 16
_NW = _NC * _NS
_RPW = _B // _NW
_T = 256
_W = _RPW * _L
_LB = 8

_OFFS = tuple(c * 16 for c in range(12)) + (184,)


@functools.partial(
    pl.kernel,
    out_type=(
        jax.ShapeDtypeStruct((_B * _L,), jnp.int32),
        jax.ShapeDtypeStruct((_B * _L,), jnp.int32),
    ),
    mesh=plsc.VectorSubcoreMesh(core_axis_name="c", subcore_axis_name="s"),
    compiler_params=pltpu.CompilerParams(
        needs_layout_passes=False, use_tc_tiling_on_sc=False),
    scratch_types=[
        pltpu.VMEM((_W,), jnp.int32),
        pltpu.VMEM((_W,), jnp.int32),
        pltpu.VMEM((1024,), jnp.int32),
        pltpu.VMEM((1024,), jnp.int32),
        pltpu.SemaphoreType.DMA,
    ],
)
def _sc_counts(src_hbm, dst_hbm, ps_hbm, pd_hbm,
               sids_v, dids_v, hs_v, hd_v, sem):
    wid = lax.axis_index("s") * _NC + lax.axis_index("c")
    base = wid * _W
    cp_s = pltpu.async_copy(src_hbm.at[pl.ds(base, _W)], sids_v, sem)
    cp_d = pltpu.async_copy(dst_hbm.at[pl.ds(base, _W)], dids_v, sem)
    cp_s.wait()
    cp_d.wait()

    zero16 = jnp.zeros((16,), jnp.int32)
    one16 = jnp.ones((16,), jnp.int32)
    iota16 = lax.broadcasted_iota(jnp.int32, (16,), 0)
    tail = iota16 >= 8

    def row_body(r, carry):
        rb = r * _L
        for k in range(64):
            hs_v[pl.ds(k * 16, 16)] = zero16
            hd_v[pl.ds(k * 16, 16)] = zero16
        for i, off in enumerate(_OFFS):
            m = tail if i == 12 else None
            plsc.addupdate_scatter(
                hs_v, [sids_v[pl.ds(rb + off, 16)]], one16, mask=m)
            plsc.addupdate_scatter(
                hd_v, [dids_v[pl.ds(rb + off, 16)]], one16, mask=m)
        for i, off in enumerate(_OFFS):
            ids = sids_v[pl.ds(rb + off, 16)]
            p = plsc.load_gather(hs_v, [ids]) * _T + plsc.load_gather(hd_v, [ids])
            p = jnp.where(ids == 0, zero16, p)
            idd = dids_v[pl.ds(rb + off, 16)]
            q = plsc.load_gather(hs_v, [idd]) * _T + plsc.load_gather(hd_v, [idd])
            q = jnp.where(idd == 0, zero16, q)
            if i < 12:
                sids_v[pl.ds(rb + off, 16)] = p
                dids_v[pl.ds(rb + off, 16)] = q
            else:
                pos = iota16 + (rb + off)
                plsc.store_scatter(sids_v, [pos], p, mask=tail)
                plsc.store_scatter(dids_v, [pos], q, mask=tail)
        return carry

    lax.fori_loop(0, _RPW, row_body, 0)

    pltpu.sync_copy(sids_v, ps_hbm.at[pl.ds(base, _W)])
    pltpu.sync_copy(dids_v, pd_hbm.at[pl.ds(base, _W)])


def _expand_kernel(ps_ref, pd_ref, w1t_ref, b1t_ref, w2_ref, b2t_ref,
                   os_ref, od_ref, t1t_ref):
    i = pl.program_id(0)

    @pl.when(i == 0)
    def _():
        a = lax.broadcasted_iota(jnp.int32, (1, _T), 1).astype(jnp.float32)
        x = w1t_ref[...] * a + b1t_ref[...]
        h = jnp.maximum(x, 0.0)
        t1t_ref[...] = lax.dot_general(
            w2_ref[...], h, (((0,), (0,)), ((), ())),
            preferred_element_type=jnp.float32,
        ) + b2t_ref[...]

    t1t = t1t_ref[...].astype(jnp.bfloat16)
    iota_v = lax.broadcasted_iota(jnp.int16, (_T, _B), 0)
    one = jnp.bfloat16(1.0)
    zero = jnp.bfloat16(0.0)
    for pair_ref, out_ref in ((ps_ref, os_ref), (pd_ref, od_ref)):
        for j in range(_LB):
            pjr = pair_ref[pl.ds(j, 1), :]
            c1 = (pjr // _T).astype(jnp.int16)
            c2 = (pjr - (pjr // _T) * _T).astype(jnp.int16)
            oh = (jnp.where(iota_v == c1, one, zero)
                  + jnp.where(iota_v == c2, one, zero))
            blk = lax.dot_general(
                t1t, oh, (((1,), (0,)), ((), ())),
                preferred_element_type=jnp.float32,
            )
            out_ref[j, :, :] = blk


def _tc_expand(ps_t, pd_t, W1, b1, W2, b2):
    return pl.pallas_call(
        _expand_kernel,
        grid=(_L // _LB,),
        in_specs=[
            pl.BlockSpec((_LB, _B), lambda i: (i, 0)),
            pl.BlockSpec((_LB, _B), lambda i: (i, 0)),
            pl.BlockSpec((_D, 1), lambda i: (0, 0)),
            pl.BlockSpec((_D, 1), lambda i: (0, 0)),
            pl.BlockSpec((_D, _D), lambda i: (0, 0)),
            pl.BlockSpec((_D, 1), lambda i: (0, 0)),
        ],
        out_specs=[
            pl.BlockSpec((_LB, _D, _B), lambda i: (i, 0, 0)),
            pl.BlockSpec((_LB, _D, _B), lambda i: (i, 0, 0)),
        ],
        out_shape=[
            jax.ShapeDtypeStruct((_L, _D, _B), jnp.float32),
            jax.ShapeDtypeStruct((_L, _D, _B), jnp.float32),
        ],
        scratch_shapes=[pltpu.VMEM((_D, _T), jnp.float32)],
    )(ps_t, pd_t, W1.reshape(_D, 1), b1.reshape(_D, 1), W2,
      b2.reshape(_D, 1))


def kernel(src_ids, dst_ids, W1, b1, W2, b2):
    ps, pd = _sc_counts(src_ids.reshape(_B * _L), dst_ids.reshape(_B * _L))
    ps_t = ps.reshape(_B, _L).T
    pd_t = pd.reshape(_B, _L).T
    os3, od3 = _tc_expand(ps_t, pd_t, W1, b1, W2, b2)
    return (jnp.transpose(os3, (2, 0, 1)), jnp.transpose(od3, (2, 0, 1)))


# speedup vs baseline: 36.2193x; 1.0402x over previous
import functools

import jax
import jax.numpy as jnp
from jax import lax
from jax.experimental import pallas as pl
from jax.experimental.pallas import tpu as pltpu
from jax.experimental.pallas import tpu_sc as plsc

_B, _L, _D = 1024, 200, 32
_NC, _NS = 2, 16
_NW = _NC * _NS
_RPW = _B // _NW
_T = 256
_TH = 208
_W = _RPW * _L
_LB = 8

_OFFS = tuple(c * 16 for c in range(12)) + (184,)


@functools.partial(
    pl.kernel,
    out_type=(
        jax.ShapeDtypeStruct((_B * _L,), jnp.int32),
        jax.ShapeDtypeStruct((_B * _L,), jnp.int32),
    ),
    mesh=plsc.VectorSubcoreMesh(core_axis_name="c", subcore_axis_name="s"),
    compiler_params=pltpu.CompilerParams(
        needs_layout_passes=False, use_tc_tiling_on_sc=False),
    scratch_types=[
        pltpu.VMEM((_W,), jnp.int32),
        pltpu.VMEM((_W,), jnp.int32),
        pltpu.VMEM((_W,), jnp.int32),
        pltpu.VMEM((_W,), jnp.int32),
        pltpu.VMEM((1024,), jnp.int32),
        pltpu.VMEM((1024,), jnp.int32),
        pltpu.SemaphoreType.DMA,
    ],
)
def _sc_counts(src_hbm, dst_hbm, ps_hbm, pd_hbm,
               sids_v, dids_v, ps_v, pd_v, hs_v, hd_v, sem):
    wid = lax.axis_index("s") * _NC + lax.axis_index("c")
    base = wid * _W
    cp_s = pltpu.async_copy(src_hbm.at[pl.ds(base, _W)], sids_v, sem)
    cp_d = pltpu.async_copy(dst_hbm.at[pl.ds(base, _W)], dids_v, sem)
    cp_s.wait()
    cp_d.wait()

    zero16 = jnp.zeros((16,), jnp.int32)
    one16 = jnp.ones((16,), jnp.int32)
    neg16 = jnp.full((16,), -1, jnp.int32)
    iota16 = lax.broadcasted_iota(jnp.int32, (16,), 0)
    tail = iota16 >= 8

    for k in range(64):
        hs_v[pl.ds(k * 16, 16)] = zero16
        hd_v[pl.ds(k * 16, 16)] = zero16

    def row_body(r, carry):
        rb = r * _L
        for i, off in enumerate(_OFFS):
            m = tail if i == 12 else None
            plsc.addupdate_scatter(
                hs_v, [sids_v[pl.ds(rb + off, 16)]], one16, mask=m)
            plsc.addupdate_scatter(
                hd_v, [dids_v[pl.ds(rb + off, 16)]], one16, mask=m)
        for off in _OFFS:
            ids = sids_v[pl.ds(rb + off, 16)]
            p = plsc.load_gather(hs_v, [ids]) * _T + plsc.load_gather(hd_v, [ids])
            ps_v[pl.ds(rb + off, 16)] = jnp.where(ids == 0, zero16, p)
            idd = dids_v[pl.ds(rb + off, 16)]
            q = plsc.load_gather(hs_v, [idd]) * _T + plsc.load_gather(hd_v, [idd])
            pd_v[pl.ds(rb + off, 16)] = jnp.where(idd == 0, zero16, q)
        for i, off in enumerate(_OFFS):
            m = tail if i == 12 else None
            plsc.addupdate_scatter(
                hs_v, [sids_v[pl.ds(rb + off, 16)]], neg16, mask=m)
            plsc.addupdate_scatter(
                hd_v, [dids_v[pl.ds(rb + off, 16)]], neg16, mask=m)
        return carry

    lax.fori_loop(0, _RPW, row_body, 0)

    pltpu.sync_copy(ps_v, ps_hbm.at[pl.ds(base, _W)])
    pltpu.sync_copy(pd_v, pd_hbm.at[pl.ds(base, _W)])


def _expand_kernel(ps_ref, pd_ref, w1t_ref, b1t_ref, w2_ref, b2t_ref,
                   os_ref, od_ref, t1t_ref):
    i = pl.program_id(0)

    @pl.when(i == 0)
    def _():
        a = lax.broadcasted_iota(jnp.int32, (1, _TH), 1).astype(jnp.float32)
        x = w1t_ref[...] * a + b1t_ref[...]
        h = jnp.maximum(x, 0.0)
        t1t_ref[...] = lax.dot_general(
            w2_ref[...], h, (((0,), (0,)), ((), ())),
            preferred_element_type=jnp.float32,
        ) + b2t_ref[...]

    t1t = t1t_ref[...].astype(jnp.bfloat16)
    iota_v = lax.broadcasted_iota(jnp.int16, (_TH, _B), 0)
    one = jnp.bfloat16(1.0)
    zero = jnp.bfloat16(0.0)
    for pair_ref, out_ref in ((ps_ref, os_ref), (pd_ref, od_ref)):
        for j in range(_LB):
            pjr = pair_ref[pl.ds(j, 1), :]
            c1 = (pjr // _T).astype(jnp.int16)
            c2 = (pjr - (pjr // _T) * _T).astype(jnp.int16)
            oh = (jnp.where(iota_v == c1, one, zero)
                  + jnp.where(iota_v == c2, one, zero))
            blk = lax.dot_general(
                t1t, oh, (((1,), (0,)), ((), ())),
                preferred_element_type=jnp.float32,
            )
            out_ref[j, :, :] = blk


def _tc_expand(ps_t, pd_t, W1, b1, W2, b2):
    return pl.pallas_call(
        _expand_kernel,
        grid=(_L // _LB,),
        in_specs=[
            pl.BlockSpec((_LB, _B), lambda i: (i, 0)),
            pl.BlockSpec((_LB, _B), lambda i: (i, 0)),
            pl.BlockSpec((_D, 1), lambda i: (0, 0)),
            pl.BlockSpec((_D, 1), lambda i: (0, 0)),
            pl.BlockSpec((_D, _D), lambda i: (0, 0)),
            pl.BlockSpec((_D, 1), lambda i: (0, 0)),
        ],
        out_specs=[
            pl.BlockSpec((_LB, _D, _B), lambda i: (i, 0, 0)),
            pl.BlockSpec((_LB, _D, _B), lambda i: (i, 0, 0)),
        ],
        out_shape=[
            jax.ShapeDtypeStruct((_L, _D, _B), jnp.float32),
            jax.ShapeDtypeStruct((_L, _D, _B), jnp.float32),
        ],
        scratch_shapes=[pltpu.VMEM((_D, _TH), jnp.float32)],
    )(ps_t, pd_t, W1.reshape(_D, 1), b1.reshape(_D, 1), W2,
      b2.reshape(_D, 1))


def kernel(src_ids, dst_ids, W1, b1, W2, b2):
    ps, pd = _sc_counts(src_ids.reshape(_B * _L), dst_ids.reshape(_B * _L))
    ps_t = ps.reshape(_B, _L).T
    pd_t = pd.reshape(_B, _L).T
    os3, od3 = _tc_expand(ps_t, pd_t, W1, b1, W2, b2)
    return (jnp.transpose(os3, (2, 0, 1)), jnp.transpose(od3, (2, 0, 1)))


# speedup vs baseline: 36.5464x; 1.0090x over previous
import functools

import jax
import jax.numpy as jnp
from jax import lax
from jax.experimental import pallas as pl
from jax.experimental.pallas import tpu as pltpu
from jax.experimental.pallas import tpu_sc as plsc

_B, _L, _D = 1024, 200, 32
_NC, _NS = 2, 16
_NW = _NC * _NS
_RPW = _B // _NW
_T = 256
_TH = 208
_W = _RPW * _L
_LB = 40

_OFFS = tuple(c * 16 for c in range(12)) + (184,)


@functools.partial(
    pl.kernel,
    out_type=(
        jax.ShapeDtypeStruct((_B * _L,), jnp.int32),
        jax.ShapeDtypeStruct((_B * _L,), jnp.int32),
    ),
    mesh=plsc.VectorSubcoreMesh(core_axis_name="c", subcore_axis_name="s"),
    compiler_params=pltpu.CompilerParams(
        needs_layout_passes=False, use_tc_tiling_on_sc=False),
    scratch_types=[
        pltpu.VMEM((_W,), jnp.int32),
        pltpu.VMEM((_W,), jnp.int32),
        pltpu.VMEM((_W,), jnp.int32),
        pltpu.VMEM((_W,), jnp.int32),
        pltpu.VMEM((1024,), jnp.int32),
        pltpu.VMEM((1024,), jnp.int32),
        pltpu.SemaphoreType.DMA,
    ],
)
def _sc_counts(src_hbm, dst_hbm, ps_hbm, pd_hbm,
               sids_v, dids_v, ps_v, pd_v, hs_v, hd_v, sem):
    wid = lax.axis_index("s") * _NC + lax.axis_index("c")
    base = wid * _W
    cp_s = pltpu.async_copy(src_hbm.at[pl.ds(base, _W)], sids_v, sem)
    cp_d = pltpu.async_copy(dst_hbm.at[pl.ds(base, _W)], dids_v, sem)
    cp_s.wait()
    cp_d.wait()

    zero16 = jnp.zeros((16,), jnp.int32)
    one16 = jnp.ones((16,), jnp.int32)
    neg16 = jnp.full((16,), -1, jnp.int32)
    iota16 = lax.broadcasted_iota(jnp.int32, (16,), 0)
    tail = iota16 >= 8

    for k in range(64):
        hs_v[pl.ds(k * 16, 16)] = zero16
        hd_v[pl.ds(k * 16, 16)] = zero16

    def row_body(r, carry):
        rb = r * _L
        for i, off in enumerate(_OFFS):
            m = tail if i == 12 else None
            plsc.addupdate_scatter(
                hs_v, [sids_v[pl.ds(rb + off, 16)]], one16, mask=m)
            plsc.addupdate_scatter(
                hd_v, [dids_v[pl.ds(rb + off, 16)]], one16, mask=m)
        for off in _OFFS:
            ids = sids_v[pl.ds(rb + off, 16)]
            p = plsc.load_gather(hs_v, [ids]) * _T + plsc.load_gather(hd_v, [ids])
            ps_v[pl.ds(rb + off, 16)] = jnp.where(ids == 0, zero16, p)
            idd = dids_v[pl.ds(rb + off, 16)]
            q = plsc.load_gather(hs_v, [idd]) * _T + plsc.load_gather(hd_v, [idd])
            pd_v[pl.ds(rb + off, 16)] = jnp.where(idd == 0, zero16, q)
        for i, off in enumerate(_OFFS):
            m = tail if i == 12 else None
            plsc.addupdate_scatter(
                hs_v, [sids_v[pl.ds(rb + off, 16)]], neg16, mask=m)
            plsc.addupdate_scatter(
                hd_v, [dids_v[pl.ds(rb + off, 16)]], neg16, mask=m)
        return carry

    lax.fori_loop(0, _RPW, row_body, 0)

    pltpu.sync_copy(ps_v, ps_hbm.at[pl.ds(base, _W)])
    pltpu.sync_copy(pd_v, pd_hbm.at[pl.ds(base, _W)])


def _expand_kernel(ps_ref, pd_ref, w1t_ref, b1t_ref, w2_ref, b2t_ref,
                   os_ref, od_ref, t1t_ref):
    i = pl.program_id(0)

    @pl.when(i == 0)
    def _():
        a = lax.broadcasted_iota(jnp.int32, (1, _TH), 1).astype(jnp.float32)
        x = w1t_ref[...] * a + b1t_ref[...]
        h = jnp.maximum(x, 0.0)
        t1t_ref[...] = lax.dot_general(
            w2_ref[...], h, (((0,), (0,)), ((), ())),
            preferred_element_type=jnp.float32,
        ) + b2t_ref[...]

    t1t = t1t_ref[...].astype(jnp.bfloat16)
    iota_v = lax.broadcasted_iota(jnp.int16, (_TH, _B), 0)
    one = jnp.bfloat16(1.0)
    zero = jnp.bfloat16(0.0)
    for pair_ref, out_ref in ((ps_ref, os_ref), (pd_ref, od_ref)):
        for j in range(_LB):
            pjr = pair_ref[pl.ds(j, 1), :]
            c1 = (pjr // _T).astype(jnp.int16)
            c2 = (pjr - (pjr // _T) * _T).astype(jnp.int16)
            oh = (jnp.where(iota_v == c1, one, zero)
                  + jnp.where(iota_v == c2, one, zero))
            blk = lax.dot_general(
                t1t, oh, (((1,), (0,)), ((), ())),
                preferred_element_type=jnp.float32,
            )
            out_ref[j, :, :] = blk


def _tc_expand(ps_t, pd_t, W1, b1, W2, b2):
    return pl.pallas_call(
        _expand_kernel,
        grid=(_L // _LB,),
        in_specs=[
            pl.BlockSpec((_LB, _B), lambda i: (i, 0)),
            pl.BlockSpec((_LB, _B), lambda i: (i, 0)),
            pl.BlockSpec((_D, 1), lambda i: (0, 0)),
            pl.BlockSpec((_D, 1), lambda i: (0, 0)),
            pl.BlockSpec((_D, _D), lambda i: (0, 0)),
            pl.BlockSpec((_D, 1), lambda i: (0, 0)),
        ],
        out_specs=[
            pl.BlockSpec((_LB, _D, _B), lambda i: (i, 0, 0)),
            pl.BlockSpec((_LB, _D, _B), lambda i: (i, 0, 0)),
        ],
        out_shape=[
            jax.ShapeDtypeStruct((_L, _D, _B), jnp.float32),
            jax.ShapeDtypeStruct((_L, _D, _B), jnp.float32),
        ],
        scratch_shapes=[pltpu.VMEM((_D, _TH), jnp.float32)],
    )(ps_t, pd_t, W1.reshape(_D, 1), b1.reshape(_D, 1), W2,
      b2.reshape(_D, 1))


def kernel(src_ids, dst_ids, W1, b1, W2, b2):
    ps, pd = _sc_counts(src_ids.reshape(_B * _L), dst_ids.reshape(_B * _L))
    ps_t = ps.reshape(_B, _L).T
    pd_t = pd.reshape(_B, _L).T
    os3, od3 = _tc_expand(ps_t, pd_t, W1, b1, W2, b2)
    return (jnp.transpose(os3, (2, 0, 1)), jnp.transpose(od3, (2, 0, 1)))
